# baseline (speedup 1.0000x reference)
import functools
import jax
import jax.numpy as jnp
from jax import lax
from jax.experimental import pallas as pl
from jax.experimental.pallas import tpu as pltpu
from jax.experimental.pallas import tpu_sc as plsc

_NC = 2
_NS = 16
_NW = _NC * _NS
_LANES = 16


def _sc_pe_add_range(x, pos_embed, seq_base, seq_rows):
    B, S, D = x.shape
    ROWS_PER_W = seq_rows // _NW
    C = min(32, ROWS_PER_W)
    CHUNKS = ROWS_PER_W // C
    NSTEPS = CHUNKS * B
    x2 = x.reshape(B * S, D)

    mesh = plsc.VectorSubcoreMesh(core_axis_name="c", subcore_axis_name="s")

    @functools.partial(
        pl.kernel,
        out_type=jax.ShapeDtypeStruct((B * seq_rows, D), jnp.float32),
        mesh=mesh,
        scratch_types=[
            pltpu.VMEM((C, D), jnp.float32),
            pltpu.VMEM((C, D), jnp.float32),
            pltpu.VMEM((C, D), jnp.float32),
            pltpu.SemaphoreType.DMA,
            pltpu.SemaphoreType.DMA,
            pltpu.SemaphoreType.DMA,
            pltpu.SemaphoreType.DMA,
            pltpu.SemaphoreType.DMA,
        ],
    )
    def sc_add(x_hbm, pe_hbm, out_hbm, peb, xb0, xb1,
               spe, sld0, sld1, sst0, sst1):
        wid = lax.axis_index("s") * _NC + lax.axis_index("c")
        seq0 = wid * ROWS_PER_W

        x_bufs = (xb0, xb1)
        ld_sems = (sld0, sld1)
        st_sems = (sst0, sst1)

        def start_pe(ci):
            cp = pltpu.make_async_copy(
                pe_hbm.at[pl.ds(seq_base + seq0 + ci * C, C)], peb, spe)
            cp.start()
            return cp

        def start_x(step):
            ci, b = divmod(step, B)
            cp = pltpu.make_async_copy(
                x_hbm.at[pl.ds(b * S + seq_base + seq0 + ci * C, C)],
                x_bufs[step % 2], ld_sems[step % 2])
            cp.start()
            return cp

        def start_store(step):
            ci, b = divmod(step, B)
            cp = pltpu.make_async_copy(
                x_bufs[step % 2],
                out_hbm.at[pl.ds(b * seq_rows + seq0 + ci * C, C)],
                st_sems[step % 2])
            cp.start()
            return cp

        pe_cp = start_pe(0)
        x_cp = [start_x(0), None]
        st_cp = [None, None]

        for step in range(NSTEPS):
            ci, b = divmod(step, B)
            par = step % 2

            if step + 1 < NSTEPS:
                if st_cp[(step + 1) % 2] is not None:
                    st_cp[(step + 1) % 2].wait()
                    st_cp[(step + 1) % 2] = None
                x_cp[(step + 1) % 2] = start_x(step + 1)

            if b == 0:
                pe_cp.wait()
            x_cp[par].wait()

            xb = x_bufs[par]

            @plsc.parallel_loop(0, C, step=1)
            def row_body(r, xb=xb):
                for c in range(0, D, _LANES):
                    sl = pl.ds(c, _LANES)
                    xb[r, sl] = xb[r, sl] + peb[r, sl]

            if b == B - 1 and ci + 1 < CHUNKS:
                pe_cp = start_pe(ci + 1)

            st_cp[par] = start_store(step)

        for p in range(2):
            if st_cp[p] is not None:
                st_cp[p].wait()

    return sc_add(x2, pos_embed).reshape(B, seq_rows, D)


def _tc_add_kernel(x_ref, pe_ref, o_ref):
    o_ref[...] = x_ref[...] + pe_ref[...]


def kernel(x, pos_embed):
    B, S, D = x.shape
    SC_ROWS = 512
    TC_ROWS = S - SC_ROWS
    BS = 1792

    tc_full = pl.pallas_call(
        _tc_add_kernel,
        grid=(TC_ROWS // BS, B),
        in_specs=[
            pl.BlockSpec((1, BS, D), lambda s, b: (b, s, 0)),
            pl.BlockSpec((BS, D), lambda s, b: (s, 0)),
        ],
        out_specs=pl.BlockSpec((1, BS, D), lambda s, b: (b, s, 0)),
        out_shape=jax.ShapeDtypeStruct(x.shape, x.dtype),
    )(x, pos_embed)

    sc_part = _sc_pe_add_range(x, pos_embed, TC_ROWS, SC_ROWS)
    return lax.dynamic_update_slice(tc_full, sc_part, (0, TC_ROWS, 0))


# speedup vs baseline: 1.5181x; 1.5181x over previous
import jax
import jax.numpy as jnp
from jax.experimental import pallas as pl


def _pe_add_kernel(x_ref, pe_ref, o_ref):
    o_ref[...] = x_ref[...] + pe_ref[...]


def kernel(x, pos_embed):
    B, S, D = x.shape
    BS = 2048
    return pl.pallas_call(
        _pe_add_kernel,
        grid=(S // BS, B),
        in_specs=[
            pl.BlockSpec((1, BS, D), lambda s, b: (b, s, 0)),
            pl.BlockSpec((BS, D), lambda s, b: (s, 0)),
        ],
        out_specs=pl.BlockSpec((1, BS, D), lambda s, b: (b, s, 0)),
        out_shape=jax.ShapeDtypeStruct(x.shape, x.dtype),
    )(x, pos_embed)


# speedup vs baseline: 1.5648x; 1.0307x over previous
import jax
import jax.numpy as jnp
from jax import lax
from jax.experimental import pallas as pl
from jax.experimental.pallas import tpu as pltpu

_R = 1024
_NBUF = 6
_PREFETCH = 3


def kernel(x, pos_embed):
    B, S, D = x.shape
    x2 = x.reshape(B * S, D)
    nstep = (B * S) // _R
    s_blocks = S // _R

    def body(x_ref, pe_ref, o_ref, pe_v, xb, ld_sems, st_sems, pe_sem):
        pe_cp = pltpu.make_async_copy(pe_ref, pe_v, pe_sem)
        pe_cp.start()

        def start_load(j):
            cp = pltpu.make_async_copy(
                x_ref.at[pl.ds(j * _R, _R)], xb.at[j % _NBUF], ld_sems.at[j])
            cp.start()
            return cp

        def start_store(j):
            cp = pltpu.make_async_copy(
                xb.at[j % _NBUF], o_ref.at[pl.ds(j * _R, _R)], st_sems.at[j])
            cp.start()
            return cp

        loads = {}
        stores = {}
        for j in range(_PREFETCH):
            loads[j] = start_load(j)
        pe_cp.wait()

        for i in range(nstep):
            j = i + _PREFETCH
            if j < nstep:
                if j - _NBUF >= 0:
                    stores.pop(j - _NBUF).wait()
                loads[j] = start_load(j)

            loads.pop(i).wait()
            b = i % _NBUF
            prow = (i % s_blocks) * _R

            def row_body(k, carry, b=b, prow=prow):
                sl = pl.ds(k * 128, 128)
                psl = pl.ds(prow + k * 128, 128)
                xb[b, sl, :] = xb[b, sl, :] + pe_v[psl, :]
                return carry

            lax.fori_loop(0, _R // 128, row_body, 0)
            stores[i] = start_store(i)

        for i in sorted(stores):
            stores[i].wait()

    out = pl.pallas_call(
        body,
        in_specs=[
            pl.BlockSpec(memory_space=pl.ANY),
            pl.BlockSpec(memory_space=pl.ANY),
        ],
        out_specs=pl.BlockSpec(memory_space=pl.ANY),
        out_shape=jax.ShapeDtypeStruct((B * S, D), x.dtype),
        scratch_shapes=[
            pltpu.VMEM((S, D), jnp.float32),
            pltpu.VMEM((_NBUF, _R, D), jnp.float32),
            pltpu.SemaphoreType.DMA(((B * S) // _R,)),
            pltpu.SemaphoreType.DMA(((B * S) // _R,)),
            pltpu.SemaphoreType.DMA,
        ],
    )(x2, pos_embed)
    return out.reshape(B, S, D)
